# baseline (speedup 1.0000x reference)
import jax
import jax.numpy as jnp
from jax.experimental import pallas as pl

_EDGE_IN = 19
_BLOCK_E = 4000


def _edge_pred_kernel(e_ref, w_ref, b_ref, o_ref):
    o_ref[...] = (
        jnp.dot(e_ref[...], w_ref[...], preferred_element_type=jnp.float32)
        + b_ref[...]
    )


def kernel(x, edge_index, e, xbatch, bn_g0, bn_b0, W00, b00, W01, b01,
           root0, rb0, bn_g1, bn_b1, W10, b10, W11, b11, root1, rb1,
           bn_g2, bn_b2, W20, b20, W21, b21, root2, rb2, Wp, bp):
    e = e.reshape(-1, _EDGE_IN)
    n_edges = e.shape[0]
    n_out = Wp.shape[1]
    block = _BLOCK_E if n_edges % _BLOCK_E == 0 else n_edges
    grid = (n_edges // block,)
    return pl.pallas_call(
        _edge_pred_kernel,
        grid=grid,
        in_specs=[
            pl.BlockSpec((block, _EDGE_IN), lambda i: (i, 0)),
            pl.BlockSpec((_EDGE_IN, n_out), lambda i: (0, 0)),
            pl.BlockSpec((1, n_out), lambda i: (0, 0)),
        ],
        out_specs=pl.BlockSpec((block, n_out), lambda i: (i, 0)),
        out_shape=jax.ShapeDtypeStruct((n_edges, n_out), jnp.float32),
    )(e, Wp, bp.reshape(1, n_out))
